# baseline (speedup 1.0000x reference)
import jax
import jax.numpy as jnp
from jax.experimental import pallas as pl
from jax.experimental.pallas import tpu as pltpu

_B = 16
_N = 16384
_K = 16
_CAGE = 8
_EPS = 1e-5

_NBLK_ENC = 8192
_NBLK_DEF = 8192
_SUB = 128
_G = 4
_LANES = _N // _SUB


def _eye3():
    r = jax.lax.broadcasted_iota(jnp.int32, (3, 3), 0)
    c = jax.lax.broadcasted_iota(jnp.int32, (3, 3), 1)
    return (r == c).astype(jnp.float32)


def _enc_kernel(pts_ref, w1_ref, b1_ref, w2_ref, b2_ref, w3_ref, b3_ref,
                lat_ref, pmin_ref, pmax_ref, ptst_ref):
    nb = pl.program_id(1)
    x = pts_ref[0]
    ptst_ref[0] = jax.lax.dot_general(_eye3(), x, (((1,), (1,)), ((), ())),
                                      preferred_element_type=jnp.float32)
    h = jax.lax.dot_general(x, w1_ref[...], (((1,), (1,)), ((), ())),
                            preferred_element_type=jnp.float32)
    h = jnp.maximum(h + b1_ref[...], 0.0)
    h = jax.lax.dot_general(h, w2_ref[...], (((1,), (1,)), ((), ())),
                            preferred_element_type=jnp.float32)
    h = jnp.maximum(h + b2_ref[...], 0.0)
    h = jax.lax.dot_general(h, w3_ref[...], (((1,), (1,)), ((), ())),
                            preferred_element_type=jnp.float32)
    h = jnp.maximum(h + b3_ref[...], 0.0)
    cur = jnp.max(h, axis=0)[None, :]
    cmin = jnp.min(x, axis=0)[None, :]
    cmax = jnp.max(x, axis=0)[None, :]

    @pl.when(nb == 0)
    def _():
        lat_ref[0] = cur
        pmin_ref[0] = cmin
        pmax_ref[0] = cmax

    @pl.when(nb != 0)
    def _():
        lat_ref[0] = jnp.maximum(lat_ref[0], cur)
        pmin_ref[0] = jnp.minimum(pmin_ref[0], cmin)
        pmax_ref[0] = jnp.maximum(pmax_ref[0], cmax)


def _fps_kernel(pts_ref, lat_ref, kw1_ref, kb1_ref, kw2_ref, kb2_ref,
                kp_ref):
    lat2 = lat_ref[0]
    h = jax.lax.dot_general(lat2, kw1_ref[...], (((1,), (1,)), ((), ())),
                            preferred_element_type=jnp.float32)
    h = jnp.maximum(h + kb1_ref[...], 0.0)
    kp2 = jax.lax.dot_general(h, kw2_ref[...], (((1,), (1,)), ((), ())),
                              preferred_element_type=jnp.float32)
    kp2 = kp2 + kb2_ref[...]

    xs = [(pts_ref[0, 3 * i], pts_ref[0, 3 * i + 1], pts_ref[0, 3 * i + 2])
          for i in range(_G)]

    iota_r = jax.lax.broadcasted_iota(jnp.int32, (_SUB, _LANES), 0)
    iota_c = jax.lax.broadcasted_iota(jnp.int32, (_SUB, _LANES), 1)
    flat = iota_r * _LANES + iota_c
    iota48 = jax.lax.broadcasted_iota(jnp.int32, (1, 3 * _K), 1)

    def seed(i):
        x, y, z = xs[i]
        mind = jnp.full((_SUB, _LANES), jnp.inf, dtype=jnp.float32)
        for k in range(_K):
            kx = kp2[i, 3 * k]
            ky = kp2[i, 3 * k + 1]
            kz = kp2[i, 3 * k + 2]
            mind = jnp.minimum(mind,
                               (x - kx) ** 2 + (y - ky) ** 2 + (z - kz) ** 2)
        return mind

    def pick(i, dist):
        x, y, z = xs[i]
        m = jnp.max(dist)
        idx = jnp.min(jnp.where(dist == m, flat, _N))
        sel = (flat == idx).astype(jnp.float32)
        return jnp.sum(sel * x), jnp.sum(sel * y), jnp.sum(sel * z)

    minds = [seed(i) for i in range(_G)]
    accs = [jnp.zeros((1, 3 * _K), jnp.float32) for _ in range(_G)]
    sels = [None] * _G
    for i in range(_G):
        sx, sy, sz = pick(i, minds[i])
        accs[i] = (accs[i] + jnp.where(iota48 == 0, sx, 0.0)
                   + jnp.where(iota48 == 1, sy, 0.0)
                   + jnp.where(iota48 == 2, sz, 0.0))
        sels[i] = (sx, sy, sz)
    for i in range(_G):
        x, y, z = xs[i]
        sx, sy, sz = sels[i]
        minds[i] = (x - sx) ** 2 + (y - sy) ** 2 + (z - sz) ** 2
    for k in range(1, _K):
        picks = [pick(i, minds[i]) for i in range(_G)]
        for i in range(_G):
            x, y, z = xs[i]
            sx, sy, sz = picks[i]
            accs[i] = (accs[i] + jnp.where(iota48 == 3 * k, sx, 0.0)
                       + jnp.where(iota48 == 3 * k + 1, sy, 0.0)
                       + jnp.where(iota48 == 3 * k + 2, sz, 0.0))
            minds[i] = jnp.minimum(
                minds[i], (x - sx) ** 2 + (y - sy) ** 2 + (z - sz) ** 2)
    kp_ref[0] = jnp.concatenate(accs, axis=0)


def _deform_kernel(pts_ref, skp_ref, tkp_ref, pmin_ref, pmax_ref,
                   cw1_ref, cb1_ref, cw2_ref, cb2_ref, out_ref, cfz_ref):
    nb = pl.program_id(1)

    @pl.when(nb == 0)
    def _():
        diff = tkp_ref[0] - skp_ref[0]
        h = jax.lax.dot_general(diff, cw1_ref[...], (((1,), (1,)), ((), ())),
                                preferred_element_type=jnp.float32)
        h = jnp.maximum(h + cb1_ref[...], 0.0)
        parts = [
            jax.lax.dot_general(cw2_ref[z * 192:(z + 1) * 192, :], h,
                                (((1,), (1,)), ((), ())),
                                preferred_element_type=jnp.float32)
            for z in range(_CAGE)
        ]
        cfz_ref[...] = jnp.concatenate(parts, axis=1) + cb2_ref[...]

    cfz = cfz_ref[...]
    xs = pts_ref[0]
    idxs = []
    ws = []
    for c in range(3):
        pmn = pmin_ref[0, 0, c]
        pmx = pmax_ref[0, 0, c]
        t = (xs[c:c + 1, :] - pmn) / (pmx - pmn + 1e-6) * (_CAGE - 1.0)
        idx = jnp.clip(t.astype(jnp.int32), 0, _CAGE - 2)
        idxs.append(idx)
        ws.append(t - idx.astype(jnp.float32))

    iota8 = jax.lax.broadcasted_iota(jnp.int32, (_CAGE, _NBLK_DEF), 0)

    def hat(idx, w):
        return (jnp.where(iota8 == idx, 1.0 - w, 0.0)
                + jnp.where(iota8 == idx + 1, w, 0.0))

    wu = hat(idxs[0], ws[0])
    wv = hat(idxs[1], ws[1])
    wz = hat(idxs[2], ws[2])
    a = jax.lax.dot_general(cfz, wz, (((1,), (0,)), ((), ())),
                            preferred_element_type=jnp.float32)
    bv = jnp.sum(a.reshape(24, 8, _NBLK_DEF) * wv[None], axis=1)
    d = jnp.sum(bv.reshape(3, 8, _NBLK_DEF) * wu[None], axis=1)
    res = xs + d
    out_ref[0] = jax.lax.dot_general(res, _eye3(), (((0,), (0,)), ((), ())),
                                     preferred_element_type=jnp.float32)


def kernel(source_points, target_points, enc_w1, enc_b1, bn1_gamma, bn1_beta,
           bn1_mean, bn1_var, enc_w2, enc_b2, bn2_gamma, bn2_beta, bn2_mean,
           bn2_var, enc_w3, enc_b3, bn3_gamma, bn3_beta, bn3_mean, bn3_var,
           kp_w1, kp_b1, kp_w2, kp_b2, cg_w1, cg_b1, cg_w2, cg_b2):
    s1 = bn1_gamma * jax.lax.rsqrt(bn1_var + _EPS)
    s2 = bn2_gamma * jax.lax.rsqrt(bn2_var + _EPS)
    s3 = bn3_gamma * jax.lax.rsqrt(bn3_var + _EPS)
    w1f = enc_w1 * s1[:, None]
    w2f = enc_w2 * s2[:, None]
    w3f = enc_w3 * s3[:, None]
    b1f = ((enc_b1 - bn1_mean) * s1 + bn1_beta)[None, :]
    b2f = ((enc_b2 - bn2_mean) * s2 + bn2_beta)[None, :]
    b3f = ((enc_b3 - bn3_mean) * s3 + bn3_beta)[None, :]

    pts_n = jnp.concatenate([source_points, target_points], axis=0)

    lat, pmin, pmax, pts_t = pl.pallas_call(
        _enc_kernel,
        grid=(2 * _B, _N // _NBLK_ENC),
        in_specs=[
            pl.BlockSpec((1, _NBLK_ENC, 3), lambda b, n: (b, n, 0)),
            pl.BlockSpec((64, 3), lambda b, n: (0, 0)),
            pl.BlockSpec((1, 64), lambda b, n: (0, 0)),
            pl.BlockSpec((128, 64), lambda b, n: (0, 0)),
            pl.BlockSpec((1, 128), lambda b, n: (0, 0)),
            pl.BlockSpec((128, 128), lambda b, n: (0, 0)),
            pl.BlockSpec((1, 128), lambda b, n: (0, 0)),
        ],
        out_specs=[
            pl.BlockSpec((1, 1, 128), lambda b, n: (b, 0, 0)),
            pl.BlockSpec((1, 1, 3), lambda b, n: (b, 0, 0)),
            pl.BlockSpec((1, 1, 3), lambda b, n: (b, 0, 0)),
            pl.BlockSpec((1, 3, _NBLK_ENC), lambda b, n: (b, 0, n)),
        ],
        out_shape=[
            jax.ShapeDtypeStruct((2 * _B, 1, 128), jnp.float32),
            jax.ShapeDtypeStruct((2 * _B, 1, 3), jnp.float32),
            jax.ShapeDtypeStruct((2 * _B, 1, 3), jnp.float32),
            jax.ShapeDtypeStruct((2 * _B, 3, _N), jnp.float32),
        ],
        compiler_params=pltpu.CompilerParams(
            dimension_semantics=("parallel", "arbitrary")),
    )(pts_n, w1f, b1f, w2f, b2f, w3f, b3f)

    nprog = 2 * _B // _G
    pts6 = pts_t.reshape(nprog, 3 * _G, _SUB, _LANES)
    lat2 = lat.reshape(nprog, _G, 128)
    kp_pair = pl.pallas_call(
        _fps_kernel,
        grid=(nprog,),
        in_specs=[
            pl.BlockSpec((1, 3 * _G, _SUB, _LANES), lambda b: (b, 0, 0, 0)),
            pl.BlockSpec((1, _G, 128), lambda b: (b, 0, 0)),
            pl.BlockSpec((128, 128), lambda b: (0, 0)),
            pl.BlockSpec((1, 128), lambda b: (0, 0)),
            pl.BlockSpec((48, 128), lambda b: (0, 0)),
            pl.BlockSpec((1, 48), lambda b: (0, 0)),
        ],
        out_specs=pl.BlockSpec((1, _G, 48), lambda b: (b, 0, 0)),
        out_shape=jax.ShapeDtypeStruct((nprog, _G, 48), jnp.float32),
        compiler_params=pltpu.CompilerParams(
            dimension_semantics=("parallel",)),
    )(pts6, lat2, kp_w1, kp_b1[None, :], kp_w2, kp_b2[None, :])
    kp_flat = kp_pair.reshape(2 * _B, 1, 48)

    lin = jnp.linspace(0.0, 1.0, _CAGE)
    grid3 = jnp.stack(jnp.meshgrid(lin, lin, lin, indexing='ij'),
                      axis=-1).reshape(_CAGE ** 3, 3)
    w2r = cg_w2.reshape(_CAGE, _CAGE, _CAGE, 3, 128).transpose(
        2, 3, 0, 1, 4).reshape(_CAGE * 192, 128)
    b2g = (cg_b2.reshape(_CAGE ** 3, 3) + grid3).reshape(
        _CAGE, _CAGE, _CAGE, 3).transpose(3, 0, 1, 2).reshape(192, _CAGE)

    deformed_t = pl.pallas_call(
        _deform_kernel,
        grid=(_B, _N // _NBLK_DEF),
        in_specs=[
            pl.BlockSpec((1, 3, _NBLK_DEF), lambda b, n: (b, 0, n)),
            pl.BlockSpec((1, 1, 48), lambda b, n: (b, 0, 0)),
            pl.BlockSpec((1, 1, 48), lambda b, n: (b + _B, 0, 0)),
            pl.BlockSpec((1, 1, 3), lambda b, n: (b, 0, 0)),
            pl.BlockSpec((1, 1, 3), lambda b, n: (b, 0, 0)),
            pl.BlockSpec((128, 48), lambda b, n: (0, 0)),
            pl.BlockSpec((1, 128), lambda b, n: (0, 0)),
            pl.BlockSpec((_CAGE * 192, 128), lambda b, n: (0, 0)),
            pl.BlockSpec((192, _CAGE), lambda b, n: (0, 0)),
        ],
        out_specs=pl.BlockSpec((1, _NBLK_DEF, 3), lambda b, n: (b, n, 0)),
        out_shape=jax.ShapeDtypeStruct((_B, _N, 3), jnp.float32),
        scratch_shapes=[pltpu.VMEM((192, 8), jnp.float32)],
        compiler_params=pltpu.CompilerParams(
            dimension_semantics=("parallel", "arbitrary")),
    )(pts_t, kp_flat, kp_flat, pmin, pmax,
      cg_w1, cg_b1[None, :], w2r, b2g)

    src_kp = kp_flat[:_B].reshape(_B, _K, 3)
    tgt_kp = kp_flat[_B:].reshape(_B, _K, 3)
    return deformed_t, src_kp, tgt_kp


# speedup vs baseline: 1.6062x; 1.6062x over previous
import jax
import jax.numpy as jnp
from jax.experimental import pallas as pl
from jax.experimental.pallas import tpu as pltpu

_B = 16
_N = 16384
_K = 16
_CAGE = 8
_EPS = 1e-5

_ECHUNK = 4096
_DCHUNK = 4096
_SUB = 128
_LANES = _N // _SUB


def _fused_kernel(src2_ref, tgt2_ref, src4_ref, tgt4_ref,
                  w1_ref, b1_ref, w2_ref, b2_ref, w3_ref, b3_ref,
                  kw1_ref, kb1_ref, kw2_ref, kb2_ref,
                  cw1_ref, cb1_ref, cw2_ref, cb2_ref,
                  out_ref, kp_ref):
    def encode(p_ref):
        lat = None
        for j in range(_N // _ECHUNK):
            x = p_ref[0, :, j * _ECHUNK:(j + 1) * _ECHUNK]
            h = jax.lax.dot_general(w1_ref[...], x,
                                    (((1,), (0,)), ((), ())),
                                    preferred_element_type=jnp.float32)
            h = jnp.maximum(h + b1_ref[...], 0.0)
            h = jax.lax.dot_general(w2_ref[...], h,
                                    (((1,), (0,)), ((), ())),
                                    preferred_element_type=jnp.float32)
            h = jnp.maximum(h + b2_ref[...], 0.0)
            h = jax.lax.dot_general(w3_ref[...], h,
                                    (((1,), (0,)), ((), ())),
                                    preferred_element_type=jnp.float32)
            h = jnp.maximum(h + b3_ref[...], 0.0)
            cur = jnp.max(h, axis=1)[None, :]
            lat = cur if lat is None else jnp.maximum(lat, cur)
        return lat

    lat2 = jnp.concatenate([encode(src2_ref), encode(tgt2_ref)], axis=0)

    h = jax.lax.dot_general(lat2, kw1_ref[...], (((1,), (1,)), ((), ())),
                            preferred_element_type=jnp.float32)
    h = jnp.maximum(h + kb1_ref[...], 0.0)
    kp2 = jax.lax.dot_general(h, kw2_ref[...], (((1,), (1,)), ((), ())),
                              preferred_element_type=jnp.float32)
    kp2 = kp2 + kb2_ref[...]

    xs = [(src4_ref[0, 0], src4_ref[0, 1], src4_ref[0, 2]),
          (tgt4_ref[0, 0], tgt4_ref[0, 1], tgt4_ref[0, 2])]

    iota_r = jax.lax.broadcasted_iota(jnp.int32, (_SUB, _LANES), 0)
    iota_c = jax.lax.broadcasted_iota(jnp.int32, (_SUB, _LANES), 1)
    flat = iota_r * _LANES + iota_c
    iota48 = jax.lax.broadcasted_iota(jnp.int32, (1, 3 * _K), 1)

    def seed(i):
        x, y, z = xs[i]
        mind = jnp.full((_SUB, _LANES), jnp.inf, dtype=jnp.float32)
        for k in range(_K):
            kx = kp2[i, 3 * k]
            ky = kp2[i, 3 * k + 1]
            kz = kp2[i, 3 * k + 2]
            mind = jnp.minimum(mind,
                               (x - kx) ** 2 + (y - ky) ** 2 + (z - kz) ** 2)
        return mind

    def pick(i, dist):
        x, y, z = xs[i]
        m = jnp.max(dist)
        idx = jnp.min(jnp.where(dist == m, flat, _N))
        sel = (flat == idx).astype(jnp.float32)
        return jnp.sum(sel * x), jnp.sum(sel * y), jnp.sum(sel * z)

    minds = [seed(0), seed(1)]
    accs = [jnp.zeros((1, 3 * _K), jnp.float32) for _ in range(2)]
    sels = [None, None]
    for i in range(2):
        sx, sy, sz = pick(i, minds[i])
        accs[i] = (accs[i] + jnp.where(iota48 == 0, sx, 0.0)
                   + jnp.where(iota48 == 1, sy, 0.0)
                   + jnp.where(iota48 == 2, sz, 0.0))
        sels[i] = (sx, sy, sz)
    for i in range(2):
        x, y, z = xs[i]
        sx, sy, sz = sels[i]
        minds[i] = (x - sx) ** 2 + (y - sy) ** 2 + (z - sz) ** 2
    for k in range(1, _K):
        picks = [pick(0, minds[0]), pick(1, minds[1])]
        for i in range(2):
            x, y, z = xs[i]
            sx, sy, sz = picks[i]
            accs[i] = (accs[i] + jnp.where(iota48 == 3 * k, sx, 0.0)
                       + jnp.where(iota48 == 3 * k + 1, sy, 0.0)
                       + jnp.where(iota48 == 3 * k + 2, sz, 0.0))
            minds[i] = jnp.minimum(
                minds[i], (x - sx) ** 2 + (y - sy) ** 2 + (z - sz) ** 2)
    kp_ref[0] = jnp.concatenate(accs, axis=0)

    diff = accs[1] - accs[0]
    hc = jax.lax.dot_general(diff, cw1_ref[...], (((1,), (1,)), ((), ())),
                             preferred_element_type=jnp.float32)
    hc = jnp.maximum(hc + cb1_ref[...], 0.0)
    parts = [
        jax.lax.dot_general(cw2_ref[z * 192:(z + 1) * 192, :], hc,
                            (((1,), (1,)), ((), ())),
                            preferred_element_type=jnp.float32)
        for z in range(_CAGE)
    ]
    cfz = jnp.concatenate(parts, axis=1) + cb2_ref[...]

    xsrc = src2_ref[0]
    pmin = jnp.min(xsrc, axis=1)
    pmax = jnp.max(xsrc, axis=1)
    iota8 = jax.lax.broadcasted_iota(jnp.int32, (_CAGE, _DCHUNK), 0)
    for j in range(_N // _DCHUNK):
        sl = slice(j * _DCHUNK, (j + 1) * _DCHUNK)
        xj = xsrc[:, sl]
        hats = []
        for c in range(3):
            t = (xj[c:c + 1, :] - pmin[c]) / (pmax[c] - pmin[c] + 1e-6) \
                * (_CAGE - 1.0)
            idx = jnp.clip(t.astype(jnp.int32), 0, _CAGE - 2)
            w = t - idx.astype(jnp.float32)
            hats.append(jnp.where(iota8 == idx, 1.0 - w, 0.0)
                        + jnp.where(iota8 == idx + 1, w, 0.0))
        wu, wv, wz = hats
        a = jax.lax.dot_general(cfz, wz, (((1,), (0,)), ((), ())),
                                preferred_element_type=jnp.float32)
        bv = jnp.sum(a.reshape(24, 8, _DCHUNK) * wv[None], axis=1)
        d = jnp.sum(bv.reshape(3, 8, _DCHUNK) * wu[None], axis=1)
        out_ref[0, :, sl] = xj + d


def kernel(source_points, target_points, enc_w1, enc_b1, bn1_gamma, bn1_beta,
           bn1_mean, bn1_var, enc_w2, enc_b2, bn2_gamma, bn2_beta, bn2_mean,
           bn2_var, enc_w3, enc_b3, bn3_gamma, bn3_beta, bn3_mean, bn3_var,
           kp_w1, kp_b1, kp_w2, kp_b2, cg_w1, cg_b1, cg_w2, cg_b2):
    s1 = bn1_gamma * jax.lax.rsqrt(bn1_var + _EPS)
    s2 = bn2_gamma * jax.lax.rsqrt(bn2_var + _EPS)
    s3 = bn3_gamma * jax.lax.rsqrt(bn3_var + _EPS)
    w1f = enc_w1 * s1[:, None]
    w2f = enc_w2 * s2[:, None]
    w3f = enc_w3 * s3[:, None]
    b1f = ((enc_b1 - bn1_mean) * s1 + bn1_beta)[:, None]
    b2f = ((enc_b2 - bn2_mean) * s2 + bn2_beta)[:, None]
    b3f = ((enc_b3 - bn3_mean) * s3 + bn3_beta)[:, None]

    src_t = source_points.transpose(0, 2, 1)
    tgt_t = target_points.transpose(0, 2, 1)
    pts_t = jnp.concatenate([src_t, tgt_t], axis=0)
    pts4 = pts_t.reshape(2 * _B, 3, _SUB, _LANES)

    lin = jnp.linspace(0.0, 1.0, _CAGE)
    grid3 = jnp.stack(jnp.meshgrid(lin, lin, lin, indexing='ij'),
                      axis=-1).reshape(_CAGE ** 3, 3)
    w2r = cg_w2.reshape(_CAGE, _CAGE, _CAGE, 3, 128).transpose(
        2, 3, 0, 1, 4).reshape(_CAGE * 192, 128)
    b2g = (cg_b2.reshape(_CAGE ** 3, 3) + grid3).reshape(
        _CAGE, _CAGE, _CAGE, 3).transpose(3, 0, 1, 2).reshape(192, _CAGE)

    deformed_t, kp_out = pl.pallas_call(
        _fused_kernel,
        grid=(_B,),
        in_specs=[
            pl.BlockSpec((1, 3, _N), lambda b: (b, 0, 0)),
            pl.BlockSpec((1, 3, _N), lambda b: (b + _B, 0, 0)),
            pl.BlockSpec((1, 3, _SUB, _LANES), lambda b: (b, 0, 0, 0)),
            pl.BlockSpec((1, 3, _SUB, _LANES), lambda b: (b + _B, 0, 0, 0)),
            pl.BlockSpec((64, 3), lambda b: (0, 0)),
            pl.BlockSpec((64, 1), lambda b: (0, 0)),
            pl.BlockSpec((128, 64), lambda b: (0, 0)),
            pl.BlockSpec((128, 1), lambda b: (0, 0)),
            pl.BlockSpec((128, 128), lambda b: (0, 0)),
            pl.BlockSpec((128, 1), lambda b: (0, 0)),
            pl.BlockSpec((128, 128), lambda b: (0, 0)),
            pl.BlockSpec((1, 128), lambda b: (0, 0)),
            pl.BlockSpec((48, 128), lambda b: (0, 0)),
            pl.BlockSpec((1, 48), lambda b: (0, 0)),
            pl.BlockSpec((128, 48), lambda b: (0, 0)),
            pl.BlockSpec((1, 128), lambda b: (0, 0)),
            pl.BlockSpec((_CAGE * 192, 128), lambda b: (0, 0)),
            pl.BlockSpec((192, _CAGE), lambda b: (0, 0)),
        ],
        out_specs=[
            pl.BlockSpec((1, 3, _N), lambda b: (b, 0, 0)),
            pl.BlockSpec((1, 2, 48), lambda b: (b, 0, 0)),
        ],
        out_shape=[
            jax.ShapeDtypeStruct((_B, 3, _N), jnp.float32),
            jax.ShapeDtypeStruct((_B, 2, 48), jnp.float32),
        ],
        compiler_params=pltpu.CompilerParams(
            dimension_semantics=("parallel",)),
    )(pts_t, pts_t, pts4, pts4,
      w1f, b1f, w2f, b2f, w3f, b3f,
      kp_w1, kp_b1[None, :], kp_w2, kp_b2[None, :],
      cg_w1, cg_b1[None, :], w2r, b2g)

    src_kp = kp_out[:, 0].reshape(_B, _K, 3)
    tgt_kp = kp_out[:, 1].reshape(_B, _K, 3)
    return deformed_t.transpose(0, 2, 1), src_kp, tgt_kp
